# Initial kernel scaffold; baseline (speedup 1.0000x reference)
import jax
import jax.numpy as jnp
from jax.experimental import pallas as pl


def kernel(x, edge_index, W, b):
    raise NotImplementedError("write your pallas kernel here")


# speedup vs baseline: 4.0326x; 4.0326x over previous
import functools

import jax
import jax.numpy as jnp
from jax import lax
from jax.experimental import pallas as pl
from jax.experimental.pallas import tpu as pltpu
from jax.experimental.pallas import tpu_sc as plsc

N = 10000
E = 320000
D = 128

DA = 136
NC = 2
NS = 16
NW = NC * NS
CH = 128
NCH = 80
EPT = CH * NCH
EPAD = EPT * NW
NACC = NS * 640


def _sc_accumulate():
    mesh = plsc.VectorSubcoreMesh(core_axis_name="c", subcore_axis_name="s")

    @functools.partial(
        pl.kernel,
        mesh=mesh,
        out_type=jax.ShapeDtypeStruct((NC, NACC, DA), jnp.float32),
        compiler_params=pltpu.CompilerParams(use_tc_tiling_on_sc=False),
        scratch_types=[
            pltpu.VMEM((NCH, CH), jnp.int32),
            pltpu.VMEM((NCH, CH), jnp.int32),
            pltpu.VMEM((CH, DA), jnp.float32),
            pltpu.VMEM_SHARED((NACC, DA), jnp.float32),
        ],
    )
    def body(xaug_hbm, src_hbm, dst_hbm, zeros_hbm, out_hbm,
             src_v, dst_v, rows_v, acc_sh):
        cid = lax.axis_index("c")
        sid = lax.axis_index("s")
        wid = sid * NC + cid

        pltpu.sync_copy(zeros_hbm, acc_sh.at[pl.ds(sid * 640, 640)])

        pltpu.sync_copy(src_hbm.at[wid], src_v)
        pltpu.sync_copy(dst_hbm.at[wid], dst_v)

        plsc.subcore_barrier()

        def chunk(j, _):
            pltpu.sync_copy(xaug_hbm.at[src_v.at[j]], rows_v)
            pltpu.sync_copy(rows_v, acc_sh.at[dst_v.at[j]], add=True)
            return _

        lax.fori_loop(0, NCH, chunk, None)

        plsc.subcore_barrier()

        pltpu.sync_copy(acc_sh.at[pl.ds(sid * 640, 640)],
                        out_hbm.at[cid, pl.ds(sid * 640, 640)])

    return body


_RB = 2000


def _tc_body(x_ref, acc_ref, w_ref, b_ref, o_ref):
    a = acc_ref[0] + acc_ref[1]
    deg = jnp.maximum(a[:, D:D + 1], 1.0)
    h = x_ref[:] + a[:, :D] / deg
    z = jnp.dot(h, w_ref[:], preferred_element_type=jnp.float32) + b_ref[:]
    o_ref[:] = jax.nn.sigmoid(z)


def kernel(x, edge_index, W, b):
    xaug = jnp.concatenate(
        [x, jnp.ones((N, 1), jnp.float32), jnp.zeros((N, DA - D - 1), jnp.float32)],
        axis=1)

    pad = EPAD - E
    src = jnp.concatenate([edge_index[0], jnp.zeros((pad,), jnp.int32)])
    dst = jnp.concatenate([edge_index[1], jnp.full((pad,), N, jnp.int32)])
    src3 = src.reshape(NW, NCH, CH)
    dst3 = dst.reshape(NW, NCH, CH)

    zeros = jnp.zeros((640, DA), jnp.float32)

    acc = _sc_accumulate()(xaug, src3, dst3, zeros)

    b2 = b.reshape(1, D)
    out = pl.pallas_call(
        _tc_body,
        grid=(N // _RB,),
        in_specs=[
            pl.BlockSpec((_RB, D), lambda i: (i, 0)),
            pl.BlockSpec((NC, _RB, DA), lambda i: (0, i, 0)),
            pl.BlockSpec((D, D), lambda i: (0, 0)),
            pl.BlockSpec((1, D), lambda i: (0, 0)),
        ],
        out_specs=pl.BlockSpec((_RB, D), lambda i: (i, 0)),
        out_shape=jax.ShapeDtypeStruct((N, D), jnp.float32),
    )(x, acc, W, b2)
    return out


# speedup vs baseline: 4.6096x; 1.1431x over previous
import functools

import jax
import jax.numpy as jnp
from jax import lax
from jax.experimental import pallas as pl
from jax.experimental.pallas import tpu as pltpu
from jax.experimental.pallas import tpu_sc as plsc

N = 10000
E = 320000
D = 128

DA = 136
NC = 2
NS = 16
NW = NC * NS
CH = 64
NCH = 160
EPT = CH * NCH
EPAD = EPT * NW
NACC = NS * 640


NBUF = 2
G = NCH // NBUF


def _sc_accumulate():
    mesh = plsc.VectorSubcoreMesh(core_axis_name="c", subcore_axis_name="s")

    @functools.partial(
        pl.kernel,
        mesh=mesh,
        out_type=jax.ShapeDtypeStruct((NC, NACC, DA), jnp.float32),
        compiler_params=pltpu.CompilerParams(use_tc_tiling_on_sc=False),
        scratch_types=[
            pltpu.VMEM((NCH, CH), jnp.int32),
            pltpu.VMEM((NCH, CH), jnp.int32),
            pltpu.VMEM((CH, DA), jnp.float32),
            pltpu.VMEM((CH, DA), jnp.float32),
            pltpu.VMEM_SHARED((NACC, DA), jnp.float32),
            pltpu.SemaphoreType.DMA,
            pltpu.SemaphoreType.DMA,
        ],
    )
    def body(xaug_hbm, src_hbm, dst_hbm, zeros_hbm, out_hbm,
             src_v, dst_v, rows0, rows1, acc_sh, sg0, sg1):
        cid = lax.axis_index("c")
        sid = lax.axis_index("s")
        wid = sid * NC + cid
        rows = (rows0, rows1)
        sg = (sg0, sg1)

        def start_g(j, b):
            pltpu.async_copy(xaug_hbm.at[src_v.at[j]], rows[b], sg[b])

        def wait_g(j, b):
            pltpu.make_async_copy(
                xaug_hbm.at[src_v.at[j]], rows[b], sg[b]).wait()

        def scatter(j, b):
            pltpu.sync_copy(rows[b], acc_sh.at[dst_v.at[j]], add=True)

        pltpu.sync_copy(zeros_hbm, acc_sh.at[pl.ds(sid * 640, 640)])

        pltpu.sync_copy(src_hbm.at[wid], src_v)
        pltpu.sync_copy(dst_hbm.at[wid], dst_v)

        plsc.subcore_barrier()

        for bb in range(NBUF):
            start_g(bb, bb)

        def group(gg, _):
            for bb in range(NBUF):
                j = gg * NBUF + bb
                wait_g(j, bb)
                scatter(j, bb)
                start_g(j + NBUF, bb)
            return _

        lax.fori_loop(0, G - 1, group, None)

        for bb in range(NBUF):
            j = (G - 1) * NBUF + bb
            wait_g(j, bb)
            scatter(j, bb)

        plsc.subcore_barrier()

        pltpu.sync_copy(acc_sh.at[pl.ds(sid * 640, 640)],
                        out_hbm.at[cid, pl.ds(sid * 640, 640)])

    return body


_RB = 2000


def _tc_body(x_ref, acc_ref, w_ref, b_ref, o_ref):
    a = acc_ref[0] + acc_ref[1]
    deg = jnp.maximum(a[:, D:D + 1], 1.0)
    h = x_ref[:] + a[:, :D] / deg
    z = jnp.dot(h, w_ref[:], preferred_element_type=jnp.float32) + b_ref[:]
    o_ref[:] = jax.nn.sigmoid(z)


def kernel(x, edge_index, W, b):
    xaug = jnp.concatenate(
        [x, jnp.ones((N, 1), jnp.float32), jnp.zeros((N, DA - D - 1), jnp.float32)],
        axis=1)

    pad = EPAD - E
    src = jnp.concatenate([edge_index[0], jnp.zeros((pad,), jnp.int32)])
    dst = jnp.concatenate([edge_index[1], jnp.full((pad,), N, jnp.int32)])
    src3 = src.reshape(NW, NCH, CH)
    dst3 = dst.reshape(NW, NCH, CH)

    zeros = jnp.zeros((640, DA), jnp.float32)

    acc = _sc_accumulate()(xaug, src3, dst3, zeros)

    b2 = b.reshape(1, D)
    out = pl.pallas_call(
        _tc_body,
        grid=(N // _RB,),
        in_specs=[
            pl.BlockSpec((_RB, D), lambda i: (i, 0)),
            pl.BlockSpec((NC, _RB, DA), lambda i: (0, i, 0)),
            pl.BlockSpec((D, D), lambda i: (0, 0)),
            pl.BlockSpec((1, D), lambda i: (0, 0)),
        ],
        out_specs=pl.BlockSpec((_RB, D), lambda i: (i, 0)),
        out_shape=jax.ShapeDtypeStruct((N, D), jnp.float32),
    )(x, acc, W, b2)
    return out
